# Initial kernel scaffold; baseline (speedup 1.0000x reference)
import jax
import jax.numpy as jnp
from jax.experimental import pallas as pl


def kernel(y, table, conv_w):
    raise NotImplementedError("write your pallas kernel here")


# speedup vs baseline: 5.2718x; 5.2718x over previous
import functools

import jax
import jax.numpy as jnp
from jax import lax
from jax.experimental import pallas as pl
from jax.experimental.pallas import tpu as pltpu
from jax.experimental.pallas import tpu_sc as plsc

EMBED = 128
CTX = 2
LANES = 16
NJ = EMBED // LANES


def _predictor_sc(y3, table, w0, w1, *, B, U):
    info = plsc.get_sparse_core_info()
    NC, NS = info.num_cores, info.num_subcores
    NW = NC * NS
    nb = B // NW
    HALF = U // 2
    NBUF = 2
    nbg = nb // NBUF

    mesh = plsc.VectorSubcoreMesh(core_axis_name="c", subcore_axis_name="s")

    @functools.partial(
        pl.kernel,
        out_type=jax.ShapeDtypeStruct((B, U, EMBED), jnp.float32),
        mesh=mesh,
        scratch_types=[
            pltpu.VMEM((NBUF, CTX, HALF), jnp.int32),
            pltpu.VMEM((NBUF, U, EMBED), jnp.float32),
            pltpu.VMEM((NBUF, U, EMBED), jnp.float32),
            pltpu.VMEM((EMBED,), jnp.float32),
            pltpu.VMEM((EMBED,), jnp.float32),
            pltpu.SemaphoreType.DMA,
            pltpu.SemaphoreType.DMA,
            pltpu.SemaphoreType.DMA,
            pltpu.SemaphoreType.DMA,
        ],
    )
    def body(y_hbm, table_hbm, w0_hbm, w1_hbm, out_hbm,
             idx_v, rows_v, outb_v, w0_v, w1_v, gs0, gs1, os0, os1):
        gsem = (gs0, gs1)
        osem = (os0, os1)
        wid = lax.axis_index("s") * NC + lax.axis_index("c")
        base = wid * nb

        pltpu.sync_copy(w0_hbm, w0_v)
        pltpu.sync_copy(w1_hbm, w1_v)
        w0r = [w0_v[pl.ds(LANES * j, LANES)] for j in range(NJ)]
        w1r = [w1_v[pl.ds(LANES * j, LANES)] for j in range(NJ)]

        def start_gather(i, buf):
            bidx = base + i
            pltpu.sync_copy(y_hbm.at[bidx], idx_v.at[buf])
            for h in range(CTX):
                pltpu.make_async_copy(
                    table_hbm.at[idx_v.at[buf, h]],
                    rows_v.at[buf, pl.ds(h * HALF, HALF)],
                    gsem[buf],
                ).start()

        def wait_gather(buf):
            for h in range(CTX):
                pltpu.make_async_copy(
                    table_hbm.at[idx_v.at[buf, h]],
                    rows_v.at[buf, pl.ds(h * HALF, HALF)],
                    gsem[buf],
                ).wait()

        def start_out(i, buf):
            pltpu.make_async_copy(
                outb_v.at[buf], out_hbm.at[base + i], osem[buf]).start()

        def wait_out(i, buf):
            pltpu.make_async_copy(
                outb_v.at[buf], out_hbm.at[base + i], osem[buf]).wait()

        def compute(buf):
            def ubody(u, carry):
                nxt = []
                for j in range(NJ):
                    t = rows_v[buf, u, pl.ds(LANES * j, LANES)]
                    outb_v[buf, u, pl.ds(LANES * j, LANES)] = (
                        w1r[j] * t + w0r[j] * carry[j])
                    nxt.append(t)
                return tuple(nxt)
            zero = jnp.zeros((LANES,), jnp.float32)
            lax.fori_loop(0, U, ubody, (zero,) * NJ)

        for buf in range(NBUF):
            start_gather(buf, buf)

        def gbody(g, _):
            for buf in range(NBUF):
                i = g * NBUF + buf
                wait_gather(buf)
                @pl.when(g >= 1)
                def _():
                    wait_out(i - NBUF, buf)
                compute(buf)
                start_out(i, buf)
                @pl.when(g < nbg - 1)
                def _():
                    start_gather(i + NBUF, buf)
            return 0

        lax.fori_loop(0, nbg, gbody, 0)
        for buf in range(NBUF):
            wait_out(nb - NBUF + buf, buf)

    return body


def kernel(y, table, conv_w):
    B, U = y.shape
    y3 = y.astype(jnp.int32).reshape(B, CTX, U // CTX)
    w0 = conv_w[:, 0, 0]
    w1 = conv_w[:, 0, 1]
    return _predictor_sc(y3, table, w0, w1, B=B, U=U)(y3, table, w0, w1)
